# baseline (speedup 1.0000x reference)
import functools

import jax
import jax.numpy as jnp
from jax.experimental import pallas as pl
from jax.experimental.pallas import tpu as pltpu

D = 2048
N_EXP = 64
TOP_K = 8
N_TOK = 16384

TB = 2048


def _router_block(x_ref, w_ref, vals_ref, inds_ref, gates_ref):
    x = x_ref[...]
    w = w_ref[...]
    logits = jax.lax.dot_general(
        x, w, (((1,), (1,)), ((), ())), preferred_element_type=jnp.float32
    )
    m = jnp.max(logits, axis=1, keepdims=True)
    e = jnp.exp(logits - m)
    s = jnp.sum(e, axis=1, keepdims=True)
    gates = e / s
    gates_ref[...] = gates

    iota = jax.lax.broadcasted_iota(jnp.int32, (TB, N_EXP), 1)
    work = gates
    vals_cols = []
    inds_cols = []
    for _ in range(TOP_K):
        mx = jnp.max(work, axis=1, keepdims=True)
        idx = jnp.min(jnp.where(work == mx, iota, N_EXP), axis=1, keepdims=True)
        vals_cols.append(mx)
        inds_cols.append(idx)
        work = jnp.where(iota == idx, -1.0, work)
    vals_ref[...] = jnp.concatenate(vals_cols, axis=1)
    inds_ref[...] = jnp.concatenate(inds_cols, axis=1)


@jax.jit
def kernel(hidden_states, W_gate, W_noise):
    del W_noise
    grid = (N_TOK // TB,)
    vals, inds, gates = pl.pallas_call(
        _router_block,
        grid=grid,
        in_specs=[
            pl.BlockSpec((TB, D), lambda i: (i, 0)),
            pl.BlockSpec((N_EXP, D), lambda i: (0, 0)),
        ],
        out_specs=[
            pl.BlockSpec((TB, TOP_K), lambda i: (i, 0)),
            pl.BlockSpec((TB, TOP_K), lambda i: (i, 0)),
            pl.BlockSpec((TB, N_EXP), lambda i: (i, 0)),
        ],
        out_shape=[
            jax.ShapeDtypeStruct((N_TOK, TOP_K), jnp.float32),
            jax.ShapeDtypeStruct((N_TOK, TOP_K), jnp.int32),
            jax.ShapeDtypeStruct((N_TOK, N_EXP), jnp.float32),
        ],
        compiler_params=pltpu.CompilerParams(
            dimension_semantics=("parallel",),
        ),
    )(hidden_states, W_gate)
    return vals, inds, gates


# speedup vs baseline: 1.3968x; 1.3968x over previous
import functools

import jax
import jax.numpy as jnp
from jax.experimental import pallas as pl
from jax.experimental.pallas import tpu as pltpu

D = 2048
N_EXP = 64
TOP_K = 8
N_TOK = 16384

TB = 2048


def _router_block(x_ref, w_ref, vals_ref, inds_ref, gates_ref):
    x = x_ref[...]
    w = w_ref[...]
    logits = jax.lax.dot_general(
        x, w, (((1,), (1,)), ((), ())), preferred_element_type=jnp.float32
    )
    lt = logits.T
    m = jnp.max(lt, axis=0, keepdims=True)
    e = jnp.exp(lt - m)
    s = jnp.sum(e, axis=0, keepdims=True)
    gt = e / s
    gates_ref[...] = gt.T

    iota = jax.lax.broadcasted_iota(jnp.int32, (N_EXP, TB), 0)
    work = gt
    vals_rows = []
    inds_rows = []
    for _ in range(TOP_K):
        mx = jnp.max(work, axis=0, keepdims=True)
        idx = jnp.min(jnp.where(work == mx, iota, N_EXP), axis=0, keepdims=True)
        vals_rows.append(mx)
        inds_rows.append(idx)
        work = jnp.where(iota == idx, -1.0, work)
    vals_ref[...] = jnp.concatenate(vals_rows, axis=0).T
    inds_ref[...] = jnp.concatenate(inds_rows, axis=0).T


@jax.jit
def kernel(hidden_states, W_gate, W_noise):
    del W_noise
    grid = (N_TOK // TB,)
    vals, inds, gates = pl.pallas_call(
        _router_block,
        grid=grid,
        in_specs=[
            pl.BlockSpec((TB, D), lambda i: (i, 0)),
            pl.BlockSpec((N_EXP, D), lambda i: (0, 0)),
        ],
        out_specs=[
            pl.BlockSpec((TB, TOP_K), lambda i: (i, 0)),
            pl.BlockSpec((TB, TOP_K), lambda i: (i, 0)),
            pl.BlockSpec((TB, N_EXP), lambda i: (i, 0)),
        ],
        out_shape=[
            jax.ShapeDtypeStruct((N_TOK, TOP_K), jnp.float32),
            jax.ShapeDtypeStruct((N_TOK, TOP_K), jnp.int32),
            jax.ShapeDtypeStruct((N_TOK, N_EXP), jnp.float32),
        ],
        compiler_params=pltpu.CompilerParams(
            dimension_semantics=("parallel",),
        ),
    )(hidden_states, W_gate)
    return vals, inds, gates
